# baseline (speedup 1.0000x reference)
import functools

import jax
import jax.numpy as jnp
from jax import lax
from jax.experimental import pallas as pl
from jax.experimental.pallas import tpu as pltpu
from jax.experimental.pallas import tpu_sc as plsc

D_MODEL = 64
SCALE = 8.0

NC = 2
NS = 16
NW = NC * NS

CHUNK = 128
LANES = 16


def _make_sc_gather(num_idx: int):
    per_w = num_idx // NW
    n_chunks = per_w // CHUNK

    mesh = plsc.VectorSubcoreMesh(core_axis_name="c", subcore_axis_name="s")

    @functools.partial(
        pl.kernel,
        out_type=jax.ShapeDtypeStruct((num_idx, D_MODEL), jnp.float32),
        mesh=mesh,
        scratch_types=[
            pltpu.VMEM((n_chunks, CHUNK), jnp.int32),
            pltpu.VMEM((CHUNK, D_MODEL), jnp.float32),
            pltpu.SemaphoreType.DMA,
        ],
        compiler_params=pltpu.CompilerParams(use_tc_tiling_on_sc=False),
    )
    def sc_kernel(idx_hbm, table_hbm, out_hbm, idx_v, rows_v, gsem):
        wid = lax.axis_index("s") * NC + lax.axis_index("c")
        pltpu.sync_copy(idx_hbm.at[wid], idx_v)
        row_base = wid * per_w

        def chunk_body(j, _):
            pltpu.async_copy(table_hbm.at[idx_v.at[j]], rows_v, gsem).wait()

            def scale_row(r, _):
                for q in range(D_MODEL // LANES):
                    sl = pl.ds(q * LANES, LANES)
                    rows_v[r, sl] = rows_v[r, sl] * SCALE
                return 0

            lax.fori_loop(0, CHUNK, scale_row, 0, unroll=2)
            pltpu.sync_copy(rows_v, out_hbm.at[pl.ds(row_base + j * CHUNK, CHUNK)])
            return 0

        lax.fori_loop(0, n_chunks, chunk_body, 0)

    return sc_kernel


def kernel(x, table):
    b, s = x.shape
    num_idx = b * s
    idx = x.reshape(NW, num_idx // (NW * CHUNK), CHUNK).astype(jnp.int32)
    out = _make_sc_gather(num_idx)(idx, table)
    return out.reshape(b, s, D_MODEL)


# speedup vs baseline: 1.1636x; 1.1636x over previous
import functools

import jax
import jax.numpy as jnp
from jax import lax
from jax.experimental import pallas as pl
from jax.experimental.pallas import tpu as pltpu
from jax.experimental.pallas import tpu_sc as plsc

D_MODEL = 64
SCALE = 8.0

NC = 2
NS = 16
NW = NC * NS

CHUNK = 128
LANES = 16
NB = 4


def _make_sc_gather(num_idx: int):
    per_w = num_idx // NW
    n_chunks = per_w // CHUNK
    n_blocks = n_chunks // NB
    assert n_chunks % NB == 0 and n_blocks >= 2

    mesh = plsc.VectorSubcoreMesh(core_axis_name="c", subcore_axis_name="s")

    @functools.partial(
        pl.kernel,
        out_type=jax.ShapeDtypeStruct((num_idx, D_MODEL), jnp.float32),
        mesh=mesh,
        scratch_types=[
            pltpu.VMEM((n_chunks, CHUNK), jnp.int32),
            pltpu.VMEM((NB, CHUNK, D_MODEL), jnp.float32),
            pltpu.VMEM((NB, CHUNK, D_MODEL), jnp.float32),
            [pltpu.SemaphoreType.DMA] * NB,
            [pltpu.SemaphoreType.DMA] * NB,
        ],
        compiler_params=pltpu.CompilerParams(use_tc_tiling_on_sc=False),
    )
    def sc_kernel(idx_hbm, table_hbm, out_hbm, idx_v, gbuf, sbuf, gsems, ssems):
        wid = lax.axis_index("s") * NC + lax.axis_index("c")
        pltpu.sync_copy(idx_hbm.at[wid], idx_v)
        row_base = wid * per_w

        def fire_gather(chunk, b):
            pltpu.async_copy(table_hbm.at[idx_v.at[chunk]], gbuf.at[b], gsems[b])

        def wait_gather(b):
            pltpu.make_async_copy(
                table_hbm.at[idx_v.at[0]], gbuf.at[b], gsems[b]).wait()

        def fire_scatter(chunk, b):
            pltpu.async_copy(
                sbuf.at[b], out_hbm.at[pl.ds(row_base + chunk * CHUNK, CHUNK)],
                ssems[b])

        def wait_scatter(b):
            pltpu.make_async_copy(
                sbuf.at[b], out_hbm.at[pl.ds(row_base, CHUNK)], ssems[b]).wait()

        def scale(b):
            g, s = gbuf.at[b], sbuf.at[b]

            @plsc.parallel_loop(0, CHUNK, unroll=4)
            def _(r):
                for q in range(D_MODEL // LANES):
                    sl = pl.ds(q * LANES, LANES)
                    s[r, sl] = g[r, sl] * SCALE

        def process(chunk, b, first, last):
            wait_gather(b)
            if not first:
                wait_scatter(b)
            scale(b)
            fire_scatter(chunk, b)
            if not last:
                fire_gather(chunk + NB, b)

        for b in range(NB):
            fire_gather(b, b)
        for b in range(NB):
            process(b, b, first=True, last=False)

        @pl.loop(1, n_blocks - 1)
        def _(j):
            base = j * NB
            for b in range(NB):
                process(base + b, b, first=False, last=False)

        for b in range(NB):
            process((n_blocks - 1) * NB + b, b, first=False, last=True)
        for b in range(NB):
            wait_scatter(b)

    return sc_kernel


def kernel(x, table):
    b, s = x.shape
    num_idx = b * s
    idx = x.reshape(NW, num_idx // (NW * CHUNK), CHUNK).astype(jnp.int32)
    out = _make_sc_gather(num_idx)(idx, table)
    return out.reshape(b, s, D_MODEL)
